# Initial kernel scaffold; baseline (speedup 1.0000x reference)
import jax
import jax.numpy as jnp
from jax.experimental import pallas as pl


def kernel(mhot_diag, mhot_med):
    raise NotImplementedError("write your pallas kernel here")


# speedup vs baseline: 31.7091x; 31.7091x over previous
import functools

import jax
import jax.numpy as jnp
from jax import lax
from jax.experimental import pallas as pl
from jax.experimental.pallas import tpu as pltpu
from jax.experimental.pallas import tpu_sc as plsc

_N_DIAG = 20000
_N_MED = 2000
_L = 16
_NS = 16
_CHUNK = 1248
_TAIL = _N_DIAG - _CHUNK * _NS
_BUF = _CHUNK + _TAIL


@functools.partial(
    pl.kernel,
    mesh=plsc.VectorSubcoreMesh(core_axis_name="c", subcore_axis_name="s"),
    out_type=jax.ShapeDtypeStruct((_N_MED,), jnp.float32),
    scratch_types=[
        pltpu.VMEM((_BUF,), jnp.int32),
        pltpu.VMEM((_L,), jnp.int32),
        pltpu.VMEM((_NS, _L), jnp.int32),
        pltpu.VMEM((_N_MED,), jnp.float32),
        pltpu.VMEM_SHARED((_NS, _L), jnp.int32),
    ],
)
def _pmd_sc(d_hbm, m_hbm, out_hbm, d_v, p_v, g_v, m_v, part_sh):
    c = lax.axis_index("c")
    s = lax.axis_index("s")

    for j in range(_TAIL // _L):
        d_v[pl.ds(_CHUNK + j * _L, _L)] = jnp.zeros((_L,), jnp.int32)
    pltpu.sync_copy(d_hbm.at[pl.ds(s * _CHUNK, _CHUNK)], d_v.at[pl.ds(0, _CHUNK)])

    @pl.when(s == 0)
    def _():
        pltpu.sync_copy(d_hbm.at[pl.ds(_CHUNK * _NS, _TAIL)],
                        d_v.at[pl.ds(_CHUNK, _TAIL)])

    one_i = jnp.ones((_L,), jnp.int32)
    zero_i = jnp.zeros((_L,), jnp.int32)
    acc = zero_i
    for j in range(_BUF // _L):
        acc = acc + jnp.where(d_v[pl.ds(j * _L, _L)] == one_i, one_i, zero_i)

    p_v[...] = acc
    pltpu.sync_copy(p_v, part_sh.at[s])
    plsc.subcore_barrier()

    @pl.when((c == 0) & (s == 0))
    def _():
        pltpu.sync_copy(part_sh, g_v)
        tot = zero_i
        for r in range(_NS):
            tot = tot + g_v[r]
        k = tot[0]
        for l in range(1, _L):
            k = k + tot[l]
        kf = k.astype(jnp.float32)
        kvec = jnp.full((_L,), kf, dtype=jnp.float32)
        zero = jnp.zeros((_L,), jnp.float32)
        one = jnp.ones((_L,), jnp.float32)
        pltpu.sync_copy(m_hbm, m_v)
        for j in range(_N_MED // _L):
            v = m_v[pl.ds(j * _L, _L)]
            m_v[pl.ds(j * _L, _L)] = jnp.minimum(jnp.maximum(v * kvec, zero), one)
        pltpu.sync_copy(m_v, out_hbm)


def kernel(mhot_diag, mhot_med):
    d = mhot_diag.reshape((_N_DIAG,))
    m = mhot_med.reshape((_N_MED,)).astype(jnp.float32)
    return _pmd_sc(d, m).reshape((1, _N_MED))


# speedup vs baseline: 33.8593x; 1.0678x over previous
import functools

import jax
import jax.numpy as jnp
from jax import lax
from jax.experimental import pallas as pl
from jax.experimental.pallas import tpu as pltpu
from jax.experimental.pallas import tpu_sc as plsc

_N_DIAG = 20000
_N_MED = 2000
_L = 16
_NS = 16
_CHUNK = 1248
_TAIL = _N_DIAG - _CHUNK * _NS
_BUF = _CHUNK + _TAIL


@functools.partial(
    pl.kernel,
    mesh=plsc.VectorSubcoreMesh(core_axis_name="c", subcore_axis_name="s",
                                num_cores=1),
    out_type=jax.ShapeDtypeStruct((_N_MED,), jnp.float32),
    scratch_types=[
        pltpu.VMEM((_BUF,), jnp.int32),
        pltpu.VMEM((_L,), jnp.int32),
        pltpu.VMEM((_NS, _L), jnp.int32),
        pltpu.VMEM((_N_MED,), jnp.float32),
        pltpu.VMEM_SHARED((_NS, _L), jnp.int32),
    ],
)
def _pmd_sc(d_hbm, m_hbm, out_hbm, d_v, p_v, g_v, m_v, part_sh):
    c = lax.axis_index("c")
    s = lax.axis_index("s")

    for j in range(_TAIL // _L):
        d_v[pl.ds(_CHUNK + j * _L, _L)] = jnp.zeros((_L,), jnp.int32)
    pltpu.sync_copy(d_hbm.at[pl.ds(s * _CHUNK, _CHUNK)], d_v.at[pl.ds(0, _CHUNK)])

    @pl.when(s == 0)
    def _():
        pltpu.sync_copy(d_hbm.at[pl.ds(_CHUNK * _NS, _TAIL)],
                        d_v.at[pl.ds(_CHUNK, _TAIL)])

    one_i = jnp.ones((_L,), jnp.int32)
    zero_i = jnp.zeros((_L,), jnp.int32)
    acc = zero_i
    for j in range(_BUF // _L):
        acc = acc + jnp.where(d_v[pl.ds(j * _L, _L)] == one_i, one_i, zero_i)

    p_v[...] = acc
    pltpu.sync_copy(p_v, part_sh.at[s])
    plsc.subcore_barrier()

    @pl.when((c == 0) & (s == 0))
    def _():
        pltpu.sync_copy(part_sh, g_v)
        tot = zero_i
        for r in range(_NS):
            tot = tot + g_v[r]
        k = tot[0]
        for l in range(1, _L):
            k = k + tot[l]
        kf = k.astype(jnp.float32)
        kvec = jnp.full((_L,), kf, dtype=jnp.float32)
        zero = jnp.zeros((_L,), jnp.float32)
        one = jnp.ones((_L,), jnp.float32)
        pltpu.sync_copy(m_hbm, m_v)
        for j in range(_N_MED // _L):
            v = m_v[pl.ds(j * _L, _L)]
            m_v[pl.ds(j * _L, _L)] = jnp.minimum(jnp.maximum(v * kvec, zero), one)
        pltpu.sync_copy(m_v, out_hbm)


def kernel(mhot_diag, mhot_med):
    d = mhot_diag.reshape((_N_DIAG,))
    m = mhot_med.reshape((_N_MED,)).astype(jnp.float32)
    return _pmd_sc(d, m).reshape((1, _N_MED))


# speedup vs baseline: 35.8100x; 1.0576x over previous
import functools

import jax
import jax.numpy as jnp
from jax import lax
from jax.experimental import pallas as pl
from jax.experimental.pallas import tpu as pltpu
from jax.experimental.pallas import tpu_sc as plsc

_N_DIAG = 20000
_N_MED = 2000
_L = 16
_NS = 16
_CHUNK = 1248
_TAIL = _N_DIAG - _CHUNK * _NS
_BUF = _CHUNK + _TAIL
_MCHUNK = 128


@functools.partial(
    pl.kernel,
    mesh=plsc.VectorSubcoreMesh(core_axis_name="c", subcore_axis_name="s",
                                num_cores=1),
    out_type=jax.ShapeDtypeStruct((_N_MED,), jnp.float32),
    scratch_types=[
        pltpu.VMEM((_BUF,), jnp.int32),
        pltpu.VMEM((_L,), jnp.int32),
        pltpu.VMEM((_NS, _L), jnp.int32),
        pltpu.VMEM((_MCHUNK,), jnp.float32),
        pltpu.VMEM_SHARED((_NS, _L), jnp.int32),
        pltpu.SemaphoreType.DMA,
    ],
)
def _pmd_sc(d_hbm, m_hbm, out_hbm, d_v, p_v, g_v, m_v, part_sh, msem):
    s = lax.axis_index("s")

    mbase = jnp.minimum(s * _MCHUNK, _N_MED - _MCHUNK)
    mcopy = pltpu.async_copy(m_hbm.at[pl.ds(mbase, _MCHUNK)], m_v, msem)

    for j in range(_TAIL // _L):
        d_v[pl.ds(_CHUNK + j * _L, _L)] = jnp.zeros((_L,), jnp.int32)
    pltpu.sync_copy(d_hbm.at[pl.ds(s * _CHUNK, _CHUNK)], d_v.at[pl.ds(0, _CHUNK)])

    @pl.when(s == 0)
    def _():
        pltpu.sync_copy(d_hbm.at[pl.ds(_CHUNK * _NS, _TAIL)],
                        d_v.at[pl.ds(_CHUNK, _TAIL)])

    one_i = jnp.ones((_L,), jnp.int32)
    zero_i = jnp.zeros((_L,), jnp.int32)
    acc = zero_i
    for j in range(_BUF // _L):
        acc = acc + jnp.where(d_v[pl.ds(j * _L, _L)] == one_i, one_i, zero_i)

    p_v[...] = acc
    pltpu.sync_copy(p_v, part_sh.at[s])
    plsc.subcore_barrier()

    pltpu.sync_copy(part_sh, g_v)
    tot = zero_i
    for r in range(_NS):
        tot = tot + g_v[r]
    k = tot[0]
    for l in range(1, _L):
        k = k + tot[l]
    kvec = jnp.full((_L,), k.astype(jnp.float32), dtype=jnp.float32)
    zero = jnp.zeros((_L,), jnp.float32)
    one = jnp.ones((_L,), jnp.float32)

    mcopy.wait()
    for j in range(_MCHUNK // _L):
        v = m_v[pl.ds(j * _L, _L)]
        m_v[pl.ds(j * _L, _L)] = jnp.minimum(jnp.maximum(v * kvec, zero), one)
    pltpu.sync_copy(m_v, out_hbm.at[pl.ds(mbase, _MCHUNK)])


def kernel(mhot_diag, mhot_med):
    d = mhot_diag.reshape((_N_DIAG,))
    m = mhot_med.reshape((_N_MED,)).astype(jnp.float32)
    return _pmd_sc(d, m).reshape((1, _N_MED))


# speedup vs baseline: 36.2364x; 1.0119x over previous
import functools

import jax
import jax.numpy as jnp
from jax import lax
from jax.experimental import pallas as pl
from jax.experimental.pallas import tpu as pltpu
from jax.experimental.pallas import tpu_sc as plsc

_N_DIAG = 20000
_N_MED = 2000
_L = 16
_NS = 16
_CHUNK = 1248
_TAIL = _N_DIAG - _CHUNK * _NS
_BUF = _CHUNK + _TAIL
_MCHUNK = 128


@functools.partial(
    pl.kernel,
    mesh=plsc.VectorSubcoreMesh(core_axis_name="c", subcore_axis_name="s",
                                num_cores=1),
    out_type=jax.ShapeDtypeStruct((_N_MED,), jnp.float32),
    scratch_types=[
        pltpu.VMEM((_BUF,), jnp.int32),
        pltpu.VMEM((_L,), jnp.int32),
        pltpu.VMEM((_NS, _L), jnp.int32),
        pltpu.VMEM((_MCHUNK,), jnp.float32),
        pltpu.VMEM_SHARED((_NS, _L), jnp.int32),
        pltpu.SemaphoreType.DMA,
        pltpu.SemaphoreType.DMA,
        pltpu.SemaphoreType.DMA,
    ],
)
def _pmd_sc(d_hbm, m_hbm, out_hbm, d_v, p_v, g_v, m_v, part_sh,
            dsem, tsem, msem):
    s = lax.axis_index("s")

    mbase = jnp.minimum(s * _MCHUNK, _N_MED - _MCHUNK)
    dcopy = pltpu.async_copy(d_hbm.at[pl.ds(s * _CHUNK, _CHUNK)],
                             d_v.at[pl.ds(0, _CHUNK)], dsem)
    tcopy = pltpu.async_copy(d_hbm.at[pl.ds(_CHUNK * _NS, _TAIL)],
                             d_v.at[pl.ds(_CHUNK, _TAIL)], tsem)
    mcopy = pltpu.async_copy(m_hbm.at[pl.ds(mbase, _MCHUNK)], m_v, msem)

    one_i = jnp.ones((_L,), jnp.int32)
    zero_i = jnp.zeros((_L,), jnp.int32)

    dcopy.wait()
    acc = zero_i
    for j in range(_CHUNK // _L):
        acc = acc + jnp.where(d_v[pl.ds(j * _L, _L)] == one_i, one_i, zero_i)

    tcopy.wait()
    tailc = zero_i
    for j in range(_CHUNK // _L, _BUF // _L):
        tailc = tailc + jnp.where(d_v[pl.ds(j * _L, _L)] == one_i, one_i, zero_i)
    is0 = jnp.where(s == 0, 1, 0)
    acc = acc + tailc * jnp.full((_L,), is0, dtype=jnp.int32)

    p_v[...] = acc
    pltpu.sync_copy(p_v, part_sh.at[s])
    plsc.subcore_barrier()

    pltpu.sync_copy(part_sh, g_v)
    tot = zero_i
    for r in range(_NS):
        tot = tot + g_v[r]
    k = tot[0]
    for l in range(1, _L):
        k = k + tot[l]
    kvec = jnp.full((_L,), k.astype(jnp.float32), dtype=jnp.float32)
    zero = jnp.zeros((_L,), jnp.float32)
    one = jnp.ones((_L,), jnp.float32)

    mcopy.wait()
    for j in range(_MCHUNK // _L):
        v = m_v[pl.ds(j * _L, _L)]
        m_v[pl.ds(j * _L, _L)] = jnp.minimum(jnp.maximum(v * kvec, zero), one)
    pltpu.sync_copy(m_v, out_hbm.at[pl.ds(mbase, _MCHUNK)])


def kernel(mhot_diag, mhot_med):
    d = mhot_diag.reshape((_N_DIAG,))
    m = mhot_med.reshape((_N_MED,)).astype(jnp.float32)
    return _pmd_sc(d, m).reshape((1, _N_MED))


# speedup vs baseline: 36.3033x; 1.0018x over previous
import functools

import jax
import jax.numpy as jnp
from jax import lax
from jax.experimental import pallas as pl
from jax.experimental.pallas import tpu as pltpu
from jax.experimental.pallas import tpu_sc as plsc

_N_DIAG = 20000
_N_MED = 2000
_L = 16
_NS = 16
_CHUNK = 1248
_TAIL = _N_DIAG - _CHUNK * _NS
_BUF = _CHUNK + _TAIL
_MCHUNK = 128


@functools.partial(
    pl.kernel,
    mesh=plsc.VectorSubcoreMesh(core_axis_name="c", subcore_axis_name="s",
                                num_cores=1),
    out_type=jax.ShapeDtypeStruct((_N_MED,), jnp.float32),
    scratch_types=[
        pltpu.VMEM((_BUF,), jnp.int32),
        pltpu.VMEM((_L,), jnp.int32),
        pltpu.VMEM((_NS, _L), jnp.int32),
        pltpu.VMEM((_MCHUNK,), jnp.float32),
        pltpu.VMEM_SHARED((_NS, _L), jnp.int32),
        pltpu.SemaphoreType.DMA,
        pltpu.SemaphoreType.DMA,
        pltpu.SemaphoreType.DMA,
    ],
)
def _pmd_sc(d_hbm, m_hbm, out_hbm, d_v, p_v, g_v, m_v, part_sh,
            dsem, tsem, msem):
    s = lax.axis_index("s")

    mbase = jnp.minimum(s * _MCHUNK, _N_MED - _MCHUNK)
    dcopy = pltpu.async_copy(d_hbm.at[pl.ds(s * _CHUNK, _CHUNK)],
                             d_v.at[pl.ds(0, _CHUNK)], dsem)
    tcopy = pltpu.async_copy(d_hbm.at[pl.ds(_CHUNK * _NS, _TAIL)],
                             d_v.at[pl.ds(_CHUNK, _TAIL)], tsem)
    mcopy = pltpu.async_copy(m_hbm.at[pl.ds(mbase, _MCHUNK)], m_v, msem)

    one_i = jnp.ones((_L,), jnp.int32)
    zero_i = jnp.zeros((_L,), jnp.int32)

    dcopy.wait()
    acc = zero_i
    for j in range(_CHUNK // _L):
        acc = acc + jnp.where(d_v[pl.ds(j * _L, _L)] == one_i, one_i, zero_i)

    tcopy.wait()
    tailc = zero_i
    for j in range(_CHUNK // _L, _BUF // _L):
        tailc = tailc + jnp.where(d_v[pl.ds(j * _L, _L)] == one_i, one_i, zero_i)
    is0 = jnp.where(s == 0, 1, 0)
    acc = acc + tailc * jnp.full((_L,), is0, dtype=jnp.int32)

    k = acc[0]
    for l in range(1, _L):
        k = k + acc[l]
    p_v[...] = jnp.full((_L,), k, dtype=jnp.int32)
    pltpu.sync_copy(p_v, part_sh.at[s])
    plsc.subcore_barrier()

    pltpu.sync_copy(part_sh, g_v)
    tot = zero_i
    for r in range(_NS):
        tot = tot + g_v[r]
    kvec = tot.astype(jnp.float32)
    zero = jnp.zeros((_L,), jnp.float32)
    one = jnp.ones((_L,), jnp.float32)

    mcopy.wait()
    for j in range(_MCHUNK // _L):
        v = m_v[pl.ds(j * _L, _L)]
        m_v[pl.ds(j * _L, _L)] = jnp.minimum(jnp.maximum(v * kvec, zero), one)
    pltpu.sync_copy(m_v, out_hbm.at[pl.ds(mbase, _MCHUNK)])


def kernel(mhot_diag, mhot_med):
    d = mhot_diag.reshape((_N_DIAG,))
    m = mhot_med.reshape((_N_MED,)).astype(jnp.float32)
    return _pmd_sc(d, m).reshape((1, _N_MED))
